# baseline (speedup 1.0000x reference)
import functools

import jax
import jax.numpy as jnp
from jax import lax
from jax.experimental import pallas as pl
from jax.experimental.pallas import tpu as pltpu
from jax.experimental.pallas import tpu_sc as plsc

_NW = 32
_CHUNK = 128
_BC = 65536
_EPS = 1e-8


def _k1_support_body(sidx_ref, tab_ref, out_ref, invn2_ref, buf, sem):
    copies = []
    for t in range(16):
        tile = pl.multiple_of((sidx_ref[t] // 128) * 128, 128)
        copies.append(pltpu.async_copy(
            tab_ref.at[:, pl.ds(tile, 128)], buf.at[t], sem))
    for c in copies:
        c.wait()
    lane = lax.broadcasted_iota(jnp.int32, (64, 128), 1)
    cols = []
    for t in range(16):
        c = sidx_ref[t] % 128
        cols.append(jnp.sum(jnp.where(lane == c, buf[t], 0.0), axis=1,
                            keepdims=True))
    sup = jnp.concatenate(cols, axis=1)
    out_ref[...] = sup
    m0, m1 = _support_means(sup)
    n2 = jnp.maximum(jnp.sqrt(jnp.sum(m0 * m0) + jnp.sum(m1 * m1)), _EPS)
    invn2_ref[...] = jnp.full((16,), 1.0 / n2, jnp.float32)


def _support_means(sup):
    t = lax.broadcasted_iota(jnp.int32, sup.shape, 1)
    m0 = jnp.sum(jnp.where((t % 2 == 0) & (t < 10), sup, 0.0), axis=1,
                 keepdims=True) * 0.2
    m1 = jnp.sum(jnp.where((t % 2 == 1) & (t < 10), sup, 0.0), axis=1,
                 keepdims=True) * 0.2
    return m0, m1


def _k2_dense_body(tab_ref, sup_ref, w0_ref, w1_ref, s_ref):
    m0, m1 = _support_means(sup_ref[...])
    x = tab_ref[...]
    mm = jnp.concatenate([m0, m1], axis=1)
    dn = (((0,), (0,)), ((), ()))
    w = lax.dot_general(mm, x, dn,
                        preferred_element_type=jnp.float32)
    ones = jnp.ones((64, 1), jnp.float32)
    s = lax.dot_general(ones, x * x, dn,
                        preferred_element_type=jnp.float32)
    w0_ref[...] = w[0]
    w1_ref[...] = w[1]
    s_ref[...] = s[0]


def _k3_gather_body(q0_hbm, q1_hbm, w0_hbm, w1_hbm, s_hbm, invn2_hbm,
                    out_hbm, *scratch):
    i0b = scratch[0:4]
    i1b = scratch[4:8]
    ga_v, gb_v, gc_v, gd_v, out_v, inv_v, sem = scratch[8:]
    wid = lax.axis_index("s") * 2 + lax.axis_index("c")
    base = wid * 512
    pltpu.sync_copy(invn2_hbm, inv_v)
    for k in range(4):
        pltpu.sync_copy(q0_hbm.at[pl.ds(base + k * _CHUNK, _CHUNK)], i0b[k])
        pltpu.sync_copy(q1_hbm.at[pl.ds(base + k * _CHUNK, _CHUNK)], i1b[k])
    copies = []
    for k in range(4):
        d = pl.ds(k * _CHUNK, _CHUNK)
        copies.append(pltpu.async_copy(w0_hbm.at[i0b[k]], ga_v.at[d], sem))
        copies.append(pltpu.async_copy(w1_hbm.at[i1b[k]], gb_v.at[d], sem))
        copies.append(pltpu.async_copy(s_hbm.at[i0b[k]], gc_v.at[d], sem))
        copies.append(pltpu.async_copy(s_hbm.at[i1b[k]], gd_v.at[d], sem))
    for c in copies:
        c.wait()
    inv_n2 = inv_v[...]
    for i in range(512 // 16):
        d = pl.ds(i * 16, 16)
        num = ga_v[d] + gb_v[d]
        sq = gc_v[d] + gd_v[d]
        bits = lax.bitcast_convert_type(sq, jnp.int32)
        y = lax.bitcast_convert_type(
            0x5F3759DF - lax.shift_right_logical(bits, 1), jnp.float32)
        for _ in range(3):
            y = y * (1.5 - 0.5 * sq * y * y)
        out_v[d] = num * y * inv_n2
    pltpu.sync_copy(out_v, out_hbm.at[pl.ds(base, 512)])


def kernel(query, support, symbol_emb):
    b = query.shape[0]
    v = symbol_emb.shape[0]
    tab_t = symbol_emb.T
    qt = query.T.astype(jnp.int32)
    q0 = qt[0]
    q1 = qt[1]
    sidx = jnp.concatenate([support.reshape(-1).astype(jnp.int32),
                            jnp.zeros((6,), jnp.int32)])
    n_blk = (v + _BC - 1) // _BC
    n_col = n_blk * _BC

    sup, invn2 = pl.pallas_call(
        _k1_support_body,
        in_specs=[pl.BlockSpec(memory_space=pltpu.SMEM),
                  pl.BlockSpec(memory_space=pl.ANY)],
        out_specs=[pl.BlockSpec((64, 16), lambda: (0, 0)),
                   pl.BlockSpec((16,), lambda: (0,))],
        out_shape=[jax.ShapeDtypeStruct((64, 16), jnp.float32),
                   jax.ShapeDtypeStruct((16,), jnp.float32)],
        scratch_shapes=[pltpu.VMEM((16, 64, 128), jnp.float32),
                        pltpu.SemaphoreType.DMA],
    )(sidx, tab_t)

    w0, w1, s = pl.pallas_call(
        _k2_dense_body,
        grid=(n_blk,),
        in_specs=[pl.BlockSpec((64, _BC), lambda i: (0, i)),
                  pl.BlockSpec((64, 16), lambda i: (0, 0))],
        out_specs=[pl.BlockSpec((_BC,), lambda i: (i,))] * 3,
        out_shape=[jax.ShapeDtypeStruct((n_col,), jnp.float32)] * 3,
        compiler_params=pltpu.CompilerParams(
            dimension_semantics=("parallel",)),
    )(tab_t, sup)

    gather = functools.partial(
        pl.kernel,
        mesh=plsc.VectorSubcoreMesh(core_axis_name="c", subcore_axis_name="s",
                                    num_cores=2),
        out_type=jax.ShapeDtypeStruct((b,), jnp.float32),
        scratch_types=tuple(
            [pltpu.VMEM((_CHUNK,), jnp.int32)] * 8
            + [pltpu.VMEM((512,), jnp.float32)] * 5
            + [pltpu.VMEM((16,), jnp.float32)]
            + [pltpu.SemaphoreType.DMA]),
    )(_k3_gather_body)
    return gather(q0, q1, w0, w1, s, invn2)
